# baseline (speedup 1.0000x reference)
import functools

import jax
import jax.numpy as jnp
from jax import lax
from jax.experimental import pallas as pl
from jax.experimental.pallas import tpu as pltpu
from jax.experimental.pallas import tpu_sc as plsc

G, H, W = 10, 30, 30
D = 1024
D4 = D // 4
L = 16
NC, NS = 2, 16
H_PER_W = 10
STRIP = W * D


def _sc_body(row_hbm, col_hbm, io_hbm, pair_hbm, out_hbm,
             buf0, buf1, row_v, col_v, io_v, pair_v, sem0, sem1):
    wid = lax.axis_index("s") * NC + lax.axis_index("c")

    @pl.when(wid < G * (H // H_PER_W))
    def _():
        g = wid // (H // H_PER_W)
        h0 = (wid % (H // H_PER_W)) * H_PER_W

        pltpu.sync_copy(row_hbm.at[pl.ds(h0 * D4, H_PER_W * D4)], row_v)
        pltpu.sync_copy(col_hbm.at[pl.ds(0, W * D4)], col_v)
        pltpu.sync_copy(io_hbm.at[pl.ds((g % 2) * D4, D4)], io_v)
        pltpu.sync_copy(pair_hbm.at[pl.ds((g // 2) * D4, D4)], pair_v)

        for buf in (buf0, buf1):
            def w_body(w, carry, buf=buf):
                base = w * D
                for c in range(D4 // L):
                    off = c * L
                    buf[pl.ds(base + D4 + off, L)] = col_v[pl.ds(w * D4 + off, L)]
                    buf[pl.ds(base + 2 * D4 + off, L)] = io_v[pl.ds(off, L)]
                    buf[pl.ds(base + 3 * D4 + off, L)] = pair_v[pl.ds(off, L)]
                return carry
            lax.fori_loop(0, W, w_body, 0)

        bufs = (buf0, buf1)
        sems = (sem0, sem1)
        copies = []
        for i in range(H_PER_W):
            buf = bufs[i % 2]
            if i >= 2:
                copies[i - 2].wait()
            def c_body(c, inner, buf=buf, i=i):
                v = row_v[pl.ds(i * D4 + c * L, L)]
                for w in range(W):
                    buf[pl.ds(w * D + c * L, L)] = v
                return inner
            lax.fori_loop(0, D4 // L, c_body, 0)
            cp = pltpu.make_async_copy(
                buf, out_hbm.at[pl.ds((g * H + h0 + i) * STRIP, STRIP)],
                sems[i % 2])
            cp.start()
            copies.append(cp)
        copies[-2].wait()
        copies[-1].wait()


_sc_call = functools.partial(
    pl.kernel,
    out_type=jax.ShapeDtypeStruct((G * H * STRIP,), jnp.float32),
    mesh=plsc.VectorSubcoreMesh(core_axis_name="c", subcore_axis_name="s"),
    scratch_types=[
        pltpu.VMEM((STRIP,), jnp.float32),
        pltpu.VMEM((STRIP,), jnp.float32),
        pltpu.VMEM((H_PER_W * D4,), jnp.float32),
        pltpu.VMEM((W * D4,), jnp.float32),
        pltpu.VMEM((D4,), jnp.float32),
        pltpu.VMEM((D4,), jnp.float32),
        pltpu.SemaphoreType.DMA,
        pltpu.SemaphoreType.DMA,
    ],
)


def kernel(x, row_table, col_table, io_table, pair_table):
    flat = _sc_call(_sc_body)(
        row_table.reshape(-1), col_table.reshape(-1),
        io_table.reshape(-1), pair_table.reshape(-1))
    return flat.reshape(G, H, W, D).astype(x.dtype)


# speedup vs baseline: 1.5051x; 1.5051x over previous
import functools

import jax
import jax.numpy as jnp
from jax import lax
from jax.experimental import pallas as pl
from jax.experimental.pallas import tpu as pltpu
from jax.experimental.pallas import tpu_sc as plsc

G, H, W = 10, 30, 30
D = 1024
D4 = D // 4
L = 16
NC, NS = 2, 16
H_PER_W = 10


def _sc_body(row_hbm, col_hbm, io_hbm, pair_hbm, out_hbm,
             buf0, buf1, row_v, col_v, io_v, pair_v, sem0, sem1):
    wid = lax.axis_index("s") * NC + lax.axis_index("c")

    @pl.when(wid < G * (H // H_PER_W))
    def _():
        g = wid // (H // H_PER_W)
        h0 = (wid % (H // H_PER_W)) * H_PER_W

        pltpu.sync_copy(row_hbm.at[pl.ds(h0 * D4, H_PER_W * D4)], row_v)
        pltpu.sync_copy(col_hbm.at[pl.ds(0, W * D4)], col_v)
        pltpu.sync_copy(io_hbm.at[pl.ds((g % 2) * D4, D4)], io_v)
        pltpu.sync_copy(pair_hbm.at[pl.ds((g // 2) * D4, D4)], pair_v)

        for buf in (buf0, buf1):
            def w_body(w, carry, buf=buf):
                for c in range(D4 // L):
                    off = c * L
                    buf[w, pl.ds(D4 + off, L)] = col_v[pl.ds(w * D4 + off, L)]
                    buf[w, pl.ds(2 * D4 + off, L)] = io_v[pl.ds(off, L)]
                    buf[w, pl.ds(3 * D4 + off, L)] = pair_v[pl.ds(off, L)]
                return carry
            lax.fori_loop(0, W, w_body, 0)

        bufs = (buf0, buf1)
        sems = (sem0, sem1)
        copies = []
        for i in range(H_PER_W):
            buf = bufs[i % 2]
            if i >= 2:
                copies[i - 2].wait()
            def c_body(c, inner, buf=buf, i=i):
                v = row_v[pl.ds(i * D4 + c * L, L)]
                def wb(w, acc, buf=buf):
                    buf[w, pl.ds(c * L, L)] = v
                    return acc
                return lax.fori_loop(0, W, wb, inner)
            lax.fori_loop(0, D4 // L, c_body, 0)
            cp = pltpu.make_async_copy(
                buf, out_hbm.at[g * H + h0 + i], sems[i % 2])
            cp.start()
            copies.append(cp)
        copies[-2].wait()
        copies[-1].wait()


_sc_call = functools.partial(
    pl.kernel,
    out_type=jax.ShapeDtypeStruct((G * H, W, D), jnp.float32),
    mesh=plsc.VectorSubcoreMesh(core_axis_name="c", subcore_axis_name="s"),
    compiler_params=pltpu.CompilerParams(use_tc_tiling_on_sc=True),
    scratch_types=[
        pltpu.VMEM((W, D), jnp.float32),
        pltpu.VMEM((W, D), jnp.float32),
        pltpu.VMEM((H_PER_W * D4,), jnp.float32),
        pltpu.VMEM((W * D4,), jnp.float32),
        pltpu.VMEM((D4,), jnp.float32),
        pltpu.VMEM((D4,), jnp.float32),
        pltpu.SemaphoreType.DMA,
        pltpu.SemaphoreType.DMA,
    ],
)


def kernel(x, row_table, col_table, io_table, pair_table):
    out3 = _sc_call(_sc_body)(
        row_table.reshape(-1), col_table.reshape(-1),
        io_table.reshape(-1), pair_table.reshape(-1))
    return out3.reshape(G, H, W, D).astype(x.dtype)


# speedup vs baseline: 5.8346x; 3.8766x over previous
import jax
import jax.numpy as jnp
from jax.experimental import pallas as pl


def _body(row_ref, col_ref, io_ref, pair_ref, out_ref, *, H, W, D4):
    p = pl.program_id(0)
    row = row_ref[:H, :]
    col = col_ref[:W, :]
    pair = pair_ref[pl.ds(p, 1), :]
    row_b = jnp.broadcast_to(row[:, None, :], (H, W, D4))
    col_b = jnp.broadcast_to(col[None, :, :], (H, W, D4))
    pair_b = jnp.broadcast_to(pair[None], (H, W, D4))
    for k in (0, 1):
        out_ref[k, :, :, 0:D4] = row_b
        out_ref[k, :, :, D4:2 * D4] = col_b
        out_ref[k, :, :, 2 * D4:3 * D4] = jnp.broadcast_to(
            io_ref[k:k + 1, :][None], (H, W, D4))
        out_ref[k, :, :, 3 * D4:4 * D4] = pair_b


def kernel(x, row_table, col_table, io_table, pair_table):
    _, G, H, W, D = x.shape
    D4 = row_table.shape[1]
    import functools
    body = functools.partial(_body, H=H, W=W, D4=D4)
    return pl.pallas_call(
        body,
        grid=(G // 2,),
        in_specs=[
            pl.BlockSpec(row_table.shape, lambda p: (0, 0)),
            pl.BlockSpec(col_table.shape, lambda p: (0, 0)),
            pl.BlockSpec(io_table.shape, lambda p: (0, 0)),
            pl.BlockSpec(pair_table.shape, lambda p: (0, 0)),
        ],
        out_specs=pl.BlockSpec((2, H, W, D), lambda p: (p, 0, 0, 0)),
        out_shape=jax.ShapeDtypeStruct((G, H, W, D), x.dtype),
    )(row_table, col_table, io_table, pair_table)
